# baseline (speedup 1.0000x reference)
import functools

import jax
import jax.numpy as jnp
from jax import lax
from jax.experimental import pallas as pl
from jax.experimental.pallas import tpu as pltpu
from jax.experimental.pallas import tpu_sc as plsc

N = 50000
E = 800000
D = 64
G = 16
L = 3

_NC = 2
_NS = 16
_NW = _NC * _NS

_F32 = jnp.float32

_EPW = E // _NW
_GCH = 128
_GFULL = _EPW // _GCH
_GTAIL = _EPW - _GFULL * _GCH


def _sc_gather_body(x_hbm, row_hbm, col_hbm, xrc_hbm,
                    idx_all, buf0, buf1, buf_t, sem0, sem1):
    c = lax.axis_index("c")
    s = lax.axis_index("s")
    wid = c * _NS + s
    base = wid * _EPW

    def do_phase(src_idx_hbm, cbase):
        pltpu.sync_copy(src_idx_hbm.at[pl.ds(base, _EPW)], idx_all)

        pltpu.async_copy(x_hbm.at[idx_all.at[pl.ds(0, _GCH)]], buf0, sem0)

        def body(j, _):
            buf_cur = j % 2
            off_n = (j + 1) * _GCH

            @pl.when(j + 1 < _GFULL)
            def _():
                @pl.when(buf_cur == 0)
                def _():
                    pltpu.async_copy(
                        x_hbm.at[idx_all.at[pl.ds(off_n, _GCH)]], buf1, sem1)

                @pl.when(buf_cur == 1)
                def _():
                    pltpu.async_copy(
                        x_hbm.at[idx_all.at[pl.ds(off_n, _GCH)]], buf0, sem0)

            off = j * _GCH

            @pl.when(buf_cur == 0)
            def _():
                pltpu.make_async_copy(x_hbm.at[idx_all.at[pl.ds(off, _GCH)]],
                                      buf0, sem0).wait()
                pltpu.sync_copy(buf0, xrc_hbm.at[pl.ds(base + off, _GCH),
                                                 pl.ds(cbase, D)])

            @pl.when(buf_cur == 1)
            def _():
                pltpu.make_async_copy(x_hbm.at[idx_all.at[pl.ds(off, _GCH)]],
                                      buf1, sem1).wait()
                pltpu.sync_copy(buf1, xrc_hbm.at[pl.ds(base + off, _GCH),
                                                 pl.ds(cbase, D)])
            return 0

        lax.fori_loop(0, _GFULL, body, 0)
        off = _GFULL * _GCH
        pltpu.async_copy(x_hbm.at[idx_all.at[pl.ds(off, _GTAIL)]],
                         buf_t, sem0).wait()
        pltpu.sync_copy(buf_t, xrc_hbm.at[pl.ds(base + off, _GTAIL),
                                          pl.ds(cbase, D)])

    do_phase(row_hbm, 0)
    do_phase(col_hbm, D)


_sc_gather = pl.kernel(
    _sc_gather_body,
    out_type=[jax.ShapeDtypeStruct((E, 2 * D), jnp.bfloat16)],
    mesh=plsc.VectorSubcoreMesh(core_axis_name="c", subcore_axis_name="s"),
    scratch_types=[pltpu.VMEM((_EPW,), jnp.int32),
                   pltpu.VMEM((_GCH, D), jnp.bfloat16),
                   pltpu.VMEM((_GCH, D), jnp.bfloat16),
                   pltpu.VMEM((_GTAIL, D), jnp.bfloat16),
                   pltpu.SemaphoreType.DMA,
                   pltpu.SemaphoreType.DMA],
    compiler_params=pltpu.CompilerParams(use_tc_tiling_on_sc=False),
)

_DH = D // 2
_EPT = E // _NS
_SCH = 128
_SFULL = _EPT // _SCH
_STAIL = _EPT - _SFULL * _SCH
_ZR = 3200
_ZR_LAST = N - (_NS - 1) * _ZR


def _sc_scatter_body(pk_hbm, col_hbm, zero_hbm, xna_hbm, xnb_hbm,
                     acc, cidx0, cidx1, mbuf0, mbuf1, cidx_t, mbuf_t,
                     csem0, csem1, msem0, msem1):
    c = lax.axis_index("c")
    s = lax.axis_index("s")

    @pl.when(s < _NS - 1)
    def _():
        pltpu.sync_copy(zero_hbm, acc.at[pl.ds(s * _ZR, _ZR)])

    @pl.when(s == _NS - 1)
    def _():
        pltpu.sync_copy(zero_hbm.at[pl.ds(0, _ZR_LAST)],
                        acc.at[pl.ds((_NS - 1) * _ZR, _ZR_LAST)])

    plsc.subcore_barrier()

    def scat(cb):
        ebase = s * _EPT

        def fire(eb, cidx, mbuf, csem, msem):
            pltpu.async_copy(col_hbm.at[pl.ds(eb, _SCH)], cidx, csem)
            pltpu.async_copy(pk_hbm.at[pl.ds(eb, _SCH), pl.ds(cb, _DH)],
                             mbuf, msem)

        def wait(eb, cidx, mbuf, csem, msem):
            pltpu.make_async_copy(col_hbm.at[pl.ds(eb, _SCH)],
                                  cidx, csem).wait()
            pltpu.make_async_copy(pk_hbm.at[pl.ds(eb, _SCH), pl.ds(cb, _DH)],
                                  mbuf, msem).wait()

        fire(ebase, cidx0, mbuf0, csem0, msem0)

        def body(j, _):
            eb = ebase + j * _SCH
            eb_n = eb + _SCH

            @pl.when(j % 2 == 0)
            def _():
                wait(eb, cidx0, mbuf0, csem0, msem0)

                @pl.when(j + 1 < _SFULL)
                def _():
                    fire(eb_n, cidx1, mbuf1, csem1, msem1)
                pltpu.sync_copy(mbuf0, acc.at[cidx0], add=True)

            @pl.when(j % 2 == 1)
            def _():
                wait(eb, cidx1, mbuf1, csem1, msem1)

                @pl.when(j + 1 < _SFULL)
                def _():
                    fire(eb_n, cidx0, mbuf0, csem0, msem0)
                pltpu.sync_copy(mbuf1, acc.at[cidx1], add=True)
            return 0

        lax.fori_loop(0, _SFULL, body, 0)
        eb = ebase + _SFULL * _SCH
        pltpu.sync_copy(col_hbm.at[pl.ds(eb, _STAIL)], cidx_t)
        pltpu.sync_copy(pk_hbm.at[pl.ds(eb, _STAIL), pl.ds(cb, _DH)], mbuf_t)
        pltpu.sync_copy(mbuf_t, acc.at[cidx_t], add=True)

    @pl.when(c == 0)
    def _():
        scat(0)

    @pl.when(c == 1)
    def _():
        scat(_DH)

    plsc.subcore_barrier()

    def wout(out_hbm):
        @pl.when(s < _NS - 1)
        def _():
            pltpu.sync_copy(acc.at[pl.ds(s * _ZR, _ZR)],
                            out_hbm.at[pl.ds(s * _ZR, _ZR)])

        @pl.when(s == _NS - 1)
        def _():
            pltpu.sync_copy(acc.at[pl.ds((_NS - 1) * _ZR, _ZR_LAST)],
                            out_hbm.at[pl.ds((_NS - 1) * _ZR, _ZR_LAST)])

    @pl.when(c == 0)
    def _():
        wout(xna_hbm)

    @pl.when(c == 1)
    def _():
        wout(xnb_hbm)


_sc_scatter = pl.kernel(
    _sc_scatter_body,
    out_type=[jax.ShapeDtypeStruct((N, _DH), _F32),
              jax.ShapeDtypeStruct((N, _DH), _F32)],
    mesh=plsc.VectorSubcoreMesh(core_axis_name="c", subcore_axis_name="s"),
    scratch_types=[pltpu.VMEM_SHARED((N, _DH), _F32),
                   pltpu.VMEM((_SCH,), jnp.int32),
                   pltpu.VMEM((_SCH,), jnp.int32),
                   pltpu.VMEM((_SCH, _DH), _F32),
                   pltpu.VMEM((_SCH, _DH), _F32),
                   pltpu.VMEM((_STAIL,), jnp.int32),
                   pltpu.VMEM((_STAIL, _DH), _F32),
                   pltpu.SemaphoreType.DMA,
                   pltpu.SemaphoreType.DMA,
                   pltpu.SemaphoreType.DMA,
                   pltpu.SemaphoreType.DMA],
    compiler_params=pltpu.CompilerParams(use_tc_tiling_on_sc=False),
)


def _sp(v):
    return jnp.maximum(v, 0.0) + jnp.log1p(jnp.exp(-jnp.abs(v)))


_BE = 8000
_BN = 5000
_BF = jnp.bfloat16


def _edge_mlp_body(xrc_ref, pk_ref, w1_ref, b1_ref, we2_ref, be2_ref,
                   wn1b_ref, wn2_ref, bn2_ref, pk_out):
    lhs = jnp.concatenate([xrc_ref[...], pk_ref[:, D:].astype(_BF)], axis=1)
    t = jnp.dot(lhs, w1_ref[...], preferred_element_type=_F32) + b1_ref[...]
    h = _sp(t[:, :D])
    ea2 = (jnp.dot(h.astype(_BF), we2_ref[...], preferred_element_type=_F32)
           + be2_ref[...])
    m = _sp(t[:, D:]
            + jnp.dot(ea2.astype(_BF), wn1b_ref[...],
                      preferred_element_type=_F32))
    msg = (jnp.dot(m.astype(_BF), wn2_ref[...], preferred_element_type=_F32)
           + bn2_ref[...])
    pk_out[...] = jnp.concatenate([msg, ea2], axis=1)


_edge_mlp = pl.pallas_call(
    _edge_mlp_body,
    grid=(E // _BE,),
    in_specs=[
        pl.BlockSpec((_BE, 2 * D), lambda i: (i, 0)),
        pl.BlockSpec((_BE, 2 * D), lambda i: (i, 0)),
        pl.BlockSpec((3 * D, 2 * D), lambda i: (0, 0)),
        pl.BlockSpec((1, 2 * D), lambda i: (0, 0)),
        pl.BlockSpec((D, D), lambda i: (0, 0)),
        pl.BlockSpec((1, D), lambda i: (0, 0)),
        pl.BlockSpec((D, D), lambda i: (0, 0)),
        pl.BlockSpec((D, D), lambda i: (0, 0)),
        pl.BlockSpec((1, D), lambda i: (0, 0)),
    ],
    out_specs=pl.BlockSpec((_BE, 2 * D), lambda i: (i, 0)),
    out_shape=jax.ShapeDtypeStruct((E, 2 * D), _F32),
)


def _stats_body(a_ref, b_ref, out_ref):
    i = pl.program_id(0)

    @pl.when(i == 0)
    def _():
        out_ref[...] = jnp.zeros_like(out_ref)

    a = a_ref[...]
    b = b_ref[...]
    srow = jnp.concatenate([jnp.sum(a, axis=0), jnp.sum(b, axis=0)])[None, :]
    qrow = jnp.concatenate([jnp.sum(a * a, axis=0),
                            jnp.sum(b * b, axis=0)])[None, :]
    out_ref[0:1, :] += srow
    out_ref[1:2, :] += qrow


_stats = pl.pallas_call(
    _stats_body,
    grid=(N // _BN,),
    in_specs=[pl.BlockSpec((_BN, _DH), lambda i: (i, 0)),
              pl.BlockSpec((_BN, _DH), lambda i: (i, 0))],
    out_specs=pl.BlockSpec((8, D), lambda i: (0, 0)),
    out_shape=jax.ShapeDtypeStruct((8, D), _F32),
)


def _norm_body(a_ref, b_ref, xp_ref, st_ref, g_ref, bt_ref, out_ref, obf_ref):
    xn = jnp.concatenate([a_ref[...], b_ref[...]], axis=1)
    mu = st_ref[0:1, :] / N
    var = st_ref[1:2, :] / N - mu * mu
    inv = lax.rsqrt(var + 1e-5)
    y = g_ref[...] * (xn - mu) * inv + bt_ref[...]
    res = _sp(y) + xp_ref[...]
    out_ref[...] = res
    obf_ref[...] = res.astype(jnp.bfloat16)


_norm = pl.pallas_call(
    _norm_body,
    grid=(N // _BN,),
    in_specs=[pl.BlockSpec((_BN, _DH), lambda i: (i, 0)),
              pl.BlockSpec((_BN, _DH), lambda i: (i, 0)),
              pl.BlockSpec((_BN, D), lambda i: (i, 0)),
              pl.BlockSpec((8, D), lambda i: (0, 0)),
              pl.BlockSpec((1, D), lambda i: (0, 0)),
              pl.BlockSpec((1, D), lambda i: (0, 0))],
    out_specs=[pl.BlockSpec((_BN, D), lambda i: (i, 0)),
               pl.BlockSpec((_BN, D), lambda i: (i, 0))],
    out_shape=[jax.ShapeDtypeStruct((N, D), _F32),
               jax.ShapeDtypeStruct((N, D), jnp.bfloat16)],
)


def _node_init_body(x_ref, bt_ref, chg_ref, wch_ref, bch_ref, wa_ref, ba_ref,
                    out_ref, obf_ref):
    ch = chg_ref[:, 0:1] * wch_ref[...] + bch_ref[...]
    chw = jnp.dot(ch, wa_ref[128:128 + 16, :],
                  preferred_element_type=_F32)
    ids = lax.broadcasted_iota(jnp.int32, (1, G), 1)
    oh = (bt_ref[...] == ids).astype(_F32)
    y = (jnp.dot(x_ref[...], wa_ref[0:128, :], preferred_element_type=_F32)
         + jnp.dot(oh, chw, preferred_element_type=_F32) + ba_ref[...])
    out_ref[...] = y
    obf_ref[...] = y.astype(jnp.bfloat16)


_node_init = pl.pallas_call(
    _node_init_body,
    grid=(N // _BN,),
    in_specs=[pl.BlockSpec((_BN, 128), lambda i: (i, 0)),
              pl.BlockSpec((_BN, 1), lambda i: (i, 0)),
              pl.BlockSpec((G, 128), lambda i: (0, 0)),
              pl.BlockSpec((1, G), lambda i: (0, 0)),
              pl.BlockSpec((1, G), lambda i: (0, 0)),
              pl.BlockSpec((144, D), lambda i: (0, 0)),
              pl.BlockSpec((1, D), lambda i: (0, 0))],
    out_specs=[pl.BlockSpec((_BN, D), lambda i: (i, 0)),
               pl.BlockSpec((_BN, D), lambda i: (i, 0))],
    out_shape=[jax.ShapeDtypeStruct((N, D), _F32),
               jax.ShapeDtypeStruct((N, D), jnp.bfloat16)],
)


def _edge_init_body(ea_ref, wb_ref, bb_ref, out_ref):
    ea0 = (jnp.dot(ea_ref[...], wb_ref[...],
                   preferred_element_type=_F32) + bb_ref[...])
    out_ref[...] = jnp.concatenate([jnp.zeros((_BE, D), _F32), ea0], axis=1)


_edge_init = pl.pallas_call(
    _edge_init_body,
    grid=(E // _BE,),
    in_specs=[pl.BlockSpec((_BE, 16), lambda i: (i, 0)),
              pl.BlockSpec((16, D), lambda i: (0, 0)),
              pl.BlockSpec((1, D), lambda i: (0, 0))],
    out_specs=pl.BlockSpec((_BE, 2 * D), lambda i: (i, 0)),
    out_shape=jax.ShapeDtypeStruct((E, 2 * D), _F32),
)


def _pool_body(x_ref, bt_ref, out_ref):
    i = pl.program_id(0)

    @pl.when(i == 0)
    def _():
        out_ref[...] = jnp.zeros_like(out_ref)

    ids = lax.broadcasted_iota(jnp.int32, (1, G), 1)
    oh = (bt_ref[...] == ids).astype(_F32)
    ones = jnp.ones((_BN, 1), _F32)
    zeros = jnp.zeros((_BN, 128 - D - 1), _F32)
    aug = jnp.concatenate([x_ref[...], ones, zeros], axis=1)
    out_ref[...] += jnp.dot(oh.T, aug, preferred_element_type=_F32)


_pool = pl.pallas_call(
    _pool_body,
    grid=(N // _BN,),
    in_specs=[pl.BlockSpec((_BN, D), lambda i: (i, 0)),
              pl.BlockSpec((_BN, 1), lambda i: (i, 0))],
    out_specs=pl.BlockSpec((G, 128), lambda i: (0, 0)),
    out_shape=jax.ShapeDtypeStruct((G, 128), _F32),
)


def _head_body(po_ref, wp1_ref, bp1_ref, wp2_ref, bp2_ref, wp3_ref, bp3_ref,
               out_ref):
    po = po_ref[...]
    gr = po[:, :D] / jnp.maximum(po[:, D:D + 1], 1.0)
    h = _sp(jnp.dot(gr, wp1_ref[...], preferred_element_type=_F32)
            + bp1_ref[...])
    h = _sp(jnp.dot(h, wp2_ref[...], preferred_element_type=_F32)
            + bp2_ref[...])
    out_ref[...] = (jnp.dot(h, wp3_ref[...], preferred_element_type=_F32)
                    + bp3_ref[...])


_head = pl.pallas_call(
    _head_body,
    grid=(1,),
    in_specs=[pl.BlockSpec((G, 128), lambda i: (0, 0)),
              pl.BlockSpec((D, 128), lambda i: (0, 0)),
              pl.BlockSpec((1, 128), lambda i: (0, 0)),
              pl.BlockSpec((128, 128), lambda i: (0, 0)),
              pl.BlockSpec((1, 128), lambda i: (0, 0)),
              pl.BlockSpec((128, 8), lambda i: (0, 0)),
              pl.BlockSpec((1, 8), lambda i: (0, 0))],
    out_specs=pl.BlockSpec((G, 8), lambda i: (0, 0)),
    out_shape=jax.ShapeDtypeStruct((G, 8), _F32),
)


def kernel(x, edge_attr, charge, params, edge_index, batch):
    p = params
    row = edge_index[0]
    col = edge_index[1]
    batch2 = batch[:, None]
    charge_b = jnp.broadcast_to(charge[:, None], (G, 128))
    zeros32 = jnp.zeros((_ZR, _DH), _F32)

    xcur, xbf = _node_init(x, batch2, charge_b, p['Wch'], p['bch'][None, :],
                           p['Wa'], p['ba'][None, :])
    pk = _edge_init(edge_attr, p['Wb'], p['bb'][None, :])

    for i in range(L):
        w1 = jnp.zeros((3 * D, 2 * D), _BF)
        w1 = w1.at[:, :D].set(p[f'We1_{i}'].astype(_BF))
        w1 = w1.at[0:D, D:].set(p[f'Wn1_{i}'][0:D].astype(_BF))
        b1 = jnp.concatenate([p[f'be1_{i}'], p[f'bn1_{i}']])[None, :]
        (xrc,) = _sc_gather(xbf, row, col)
        pk = _edge_mlp(
            xrc, pk, w1, b1,
            p[f'We2_{i}'].astype(_BF), p[f'be2_{i}'][None, :],
            p[f'Wn1_{i}'][D:2 * D].astype(_BF),
            p[f'Wn2_{i}'].astype(_BF), p[f'bn2_{i}'][None, :])
        xna, xnb = _sc_scatter(pk, col, zeros32)
        st = _stats(xna, xnb)
        xcur, xbf = _norm(xna, xnb, xcur, st,
                          p[f'g_{i}'][None, :], p[f'bt_{i}'][None, :])

    po = _pool(xcur, batch2)
    out = _head(po, p['Wp1'], p['bp1'][None, :], p['Wp2'], p['bp2'][None, :],
                jnp.pad(p['Wp3'], ((0, 0), (0, 7))),
                jnp.pad(p['bp3'][None, :], ((0, 0), (0, 7))))
    return out[:, 0]


# speedup vs baseline: 1.6287x; 1.6287x over previous
import functools

import jax
import jax.numpy as jnp
from jax import lax
from jax.experimental import pallas as pl
from jax.experimental.pallas import tpu as pltpu
from jax.experimental.pallas import tpu_sc as plsc

N = 50000
E = 800000
D = 64
G = 16
L = 3

_NC = 2
_NS = 16
_NW = _NC * _NS

_F32 = jnp.float32

E2 = E // 2
_GCH = 128
_GNCH = E2 // _GCH
_GBASE = _GNCH // _NW
_GEXTRA = _GNCH - _GBASE * _NW


def _sc_gather_body(x_hbm, row_hbm, col_hbm, xrc_hbm,
                    idx0, idx1, buf0, buf1, isem0, isem1, sem0, sem1):
    c = lax.axis_index("c")
    s = lax.axis_index("s")
    wid = c * _NS + s
    nk = _GBASE + jnp.where(wid < _GEXTRA, 1, 0)

    def do_phase(src_idx_hbm, cbase):
        def fire_idx(k, idx, isem):
            eb = (wid + k * _NW) * _GCH
            pltpu.async_copy(src_idx_hbm.at[pl.ds(eb, _GCH)], idx, isem)

        def wait_idx(k, idx, isem):
            eb = (wid + k * _NW) * _GCH
            pltpu.make_async_copy(src_idx_hbm.at[pl.ds(eb, _GCH)],
                                  idx, isem).wait()

        fire_idx(0, idx0, isem0)

        def body(k, _):
            eb = (wid + k * _NW) * _GCH

            @pl.when(k % 2 == 0)
            def _():
                wait_idx(k, idx0, isem0)

                @pl.when(k + 1 < nk)
                def _():
                    fire_idx(k + 1, idx1, isem1)
                pltpu.async_copy(x_hbm.at[idx0], buf0, sem0).wait()
                pltpu.sync_copy(buf0, xrc_hbm.at[pl.ds(eb, _GCH),
                                                 pl.ds(cbase, D)])

            @pl.when(k % 2 == 1)
            def _():
                wait_idx(k, idx1, isem1)

                @pl.when(k + 1 < nk)
                def _():
                    fire_idx(k + 1, idx0, isem0)
                pltpu.async_copy(x_hbm.at[idx1], buf1, sem1).wait()
                pltpu.sync_copy(buf1, xrc_hbm.at[pl.ds(eb, _GCH),
                                                 pl.ds(cbase, D)])
            return 0

        lax.fori_loop(0, nk, body, 0)

    do_phase(row_hbm, 0)
    do_phase(col_hbm, D)


_sc_gather = pl.kernel(
    _sc_gather_body,
    out_type=[jax.ShapeDtypeStruct((E2, 2 * D), _F32)],
    mesh=plsc.VectorSubcoreMesh(core_axis_name="c", subcore_axis_name="s"),
    scratch_types=[pltpu.VMEM((_GCH,), jnp.int32),
                   pltpu.VMEM((_GCH,), jnp.int32),
                   pltpu.VMEM((_GCH, D), _F32),
                   pltpu.VMEM((_GCH, D), _F32),
                   pltpu.SemaphoreType.DMA,
                   pltpu.SemaphoreType.DMA,
                   pltpu.SemaphoreType.DMA,
                   pltpu.SemaphoreType.DMA],
    compiler_params=pltpu.CompilerParams(use_tc_tiling_on_sc=False),
)

_DH = D // 2
_EPT = E2 // _NS
_SCH = 128
_SFULL = _EPT // _SCH
_STAIL = _EPT - _SFULL * _SCH
_ZR = 3200
_ZR_LAST = N - (_NS - 1) * _ZR


def _sc_scatter_body(pk_hbm, col_hbm, zero_hbm, xn_hbm,
                     acc, cidx0, cidx1, mbuf0, mbuf1, cidx_t, mbuf_t,
                     csem0, csem1, msem0, msem1):
    c = lax.axis_index("c")
    s = lax.axis_index("s")

    @pl.when(s < _NS - 1)
    def _():
        pltpu.sync_copy(zero_hbm, acc.at[pl.ds(s * _ZR, _ZR)])

    @pl.when(s == _NS - 1)
    def _():
        pltpu.sync_copy(zero_hbm.at[pl.ds(0, _ZR_LAST)],
                        acc.at[pl.ds((_NS - 1) * _ZR, _ZR_LAST)])

    plsc.subcore_barrier()

    def scat(cb):
        ebase = s * _EPT

        def fire(eb, cidx, mbuf, csem, msem):
            pltpu.async_copy(col_hbm.at[pl.ds(eb, _SCH)], cidx, csem)
            pltpu.async_copy(pk_hbm.at[pl.ds(eb, _SCH), pl.ds(cb, _DH)],
                             mbuf, msem)

        def wait(eb, cidx, mbuf, csem, msem):
            pltpu.make_async_copy(col_hbm.at[pl.ds(eb, _SCH)],
                                  cidx, csem).wait()
            pltpu.make_async_copy(pk_hbm.at[pl.ds(eb, _SCH), pl.ds(cb, _DH)],
                                  mbuf, msem).wait()

        fire(ebase, cidx0, mbuf0, csem0, msem0)

        def body(j, _):
            eb = ebase + j * _SCH
            eb_n = eb + _SCH

            @pl.when(j % 2 == 0)
            def _():
                wait(eb, cidx0, mbuf0, csem0, msem0)

                @pl.when(j + 1 < _SFULL)
                def _():
                    fire(eb_n, cidx1, mbuf1, csem1, msem1)
                pltpu.sync_copy(mbuf0, acc.at[cidx0], add=True)

            @pl.when(j % 2 == 1)
            def _():
                wait(eb, cidx1, mbuf1, csem1, msem1)

                @pl.when(j + 1 < _SFULL)
                def _():
                    fire(eb_n, cidx0, mbuf0, csem0, msem0)
                pltpu.sync_copy(mbuf1, acc.at[cidx1], add=True)
            return 0

        lax.fori_loop(0, _SFULL, body, 0)
        eb = ebase + _SFULL * _SCH
        pltpu.sync_copy(col_hbm.at[pl.ds(eb, _STAIL)], cidx_t)
        pltpu.sync_copy(pk_hbm.at[pl.ds(eb, _STAIL), pl.ds(cb, _DH)], mbuf_t)
        pltpu.sync_copy(mbuf_t, acc.at[cidx_t], add=True)

    @pl.when(c == 0)
    def _():
        scat(0)

    @pl.when(c == 1)
    def _():
        scat(_DH)

    plsc.subcore_barrier()

    def wout(cb):
        @pl.when(s < _NS - 1)
        def _():
            pltpu.sync_copy(acc.at[pl.ds(s * _ZR, _ZR)],
                            xn_hbm.at[pl.ds(s * _ZR, _ZR), pl.ds(cb, _DH)])

        @pl.when(s == _NS - 1)
        def _():
            pltpu.sync_copy(acc.at[pl.ds((_NS - 1) * _ZR, _ZR_LAST)],
                            xn_hbm.at[pl.ds((_NS - 1) * _ZR, _ZR_LAST),
                                      pl.ds(cb, _DH)])

    @pl.when(c == 0)
    def _():
        wout(0)

    @pl.when(c == 1)
    def _():
        wout(_DH)


_sc_scatter = pl.kernel(
    _sc_scatter_body,
    out_type=[jax.ShapeDtypeStruct((N, 2 * D), _F32)],
    mesh=plsc.VectorSubcoreMesh(core_axis_name="c", subcore_axis_name="s"),
    scratch_types=[pltpu.VMEM_SHARED((N, _DH), _F32),
                   pltpu.VMEM((_SCH,), jnp.int32),
                   pltpu.VMEM((_SCH,), jnp.int32),
                   pltpu.VMEM((_SCH, _DH), _F32),
                   pltpu.VMEM((_SCH, _DH), _F32),
                   pltpu.VMEM((_STAIL,), jnp.int32),
                   pltpu.VMEM((_STAIL, _DH), _F32),
                   pltpu.SemaphoreType.DMA,
                   pltpu.SemaphoreType.DMA,
                   pltpu.SemaphoreType.DMA,
                   pltpu.SemaphoreType.DMA],
    compiler_params=pltpu.CompilerParams(use_tc_tiling_on_sc=False),
)


def _sp(v):
    return jnp.maximum(v, 0.0) + jnp.log1p(jnp.exp(-jnp.abs(v)))


_BE = 8000
_BN = 5000
_BF = jnp.bfloat16


def _edge_mlp_body(xrc_ref, pk_ref, w1_ref, b1_ref, we2_ref, be2_ref,
                   wn1b_ref, wn2_ref, bn2_ref, pk_out):
    lhs = jnp.concatenate([xrc_ref[...], pk_ref[:, D:]], axis=1).astype(_BF)
    t = jnp.dot(lhs, w1_ref[...], preferred_element_type=_F32) + b1_ref[...]
    h = _sp(t[:, :D])
    ea2 = (jnp.dot(h.astype(_BF), we2_ref[...], preferred_element_type=_F32)
           + be2_ref[...])
    m = _sp(t[:, D:]
            + jnp.dot(ea2.astype(_BF), wn1b_ref[...],
                      preferred_element_type=_F32))
    msg = (jnp.dot(m.astype(_BF), wn2_ref[...], preferred_element_type=_F32)
           + bn2_ref[...])
    pk_out[...] = jnp.concatenate([msg, ea2], axis=1)


_edge_mlp = pl.pallas_call(
    _edge_mlp_body,
    grid=(E2 // _BE,),
    in_specs=[
        pl.BlockSpec((_BE, 2 * D), lambda i: (i, 0)),
        pl.BlockSpec((_BE, 2 * D), lambda i: (i, 0)),
        pl.BlockSpec((3 * D, 2 * D), lambda i: (0, 0)),
        pl.BlockSpec((1, 2 * D), lambda i: (0, 0)),
        pl.BlockSpec((D, D), lambda i: (0, 0)),
        pl.BlockSpec((1, D), lambda i: (0, 0)),
        pl.BlockSpec((D, D), lambda i: (0, 0)),
        pl.BlockSpec((D, D), lambda i: (0, 0)),
        pl.BlockSpec((1, D), lambda i: (0, 0)),
    ],
    out_specs=pl.BlockSpec((_BE, 2 * D), lambda i: (i, 0)),
    out_shape=jax.ShapeDtypeStruct((E2, 2 * D), _F32),
)


def _stats_body(a_ref, b_ref, out_ref):
    i = pl.program_id(0)

    @pl.when(i == 0)
    def _():
        out_ref[...] = jnp.zeros_like(out_ref)

    xn = a_ref[:, :D] + b_ref[:, :D]
    out_ref[0:1, :] += jnp.sum(xn, axis=0)[None, :]
    out_ref[1:2, :] += jnp.sum(xn * xn, axis=0)[None, :]


_stats = pl.pallas_call(
    _stats_body,
    grid=(N // _BN,),
    in_specs=[pl.BlockSpec((_BN, 2 * D), lambda i: (i, 0)),
              pl.BlockSpec((_BN, 2 * D), lambda i: (i, 0))],
    out_specs=pl.BlockSpec((8, D), lambda i: (0, 0)),
    out_shape=jax.ShapeDtypeStruct((8, D), _F32),
)


def _norm_body(a_ref, b_ref, xp_ref, st_ref, g_ref, bt_ref, out_ref):
    xn = a_ref[:, :D] + b_ref[:, :D]
    mu = st_ref[0:1, :] / N
    var = st_ref[1:2, :] / N - mu * mu
    inv = lax.rsqrt(var + 1e-5)
    y = g_ref[...] * (xn - mu) * inv + bt_ref[...]
    out_ref[...] = _sp(y) + xp_ref[...]


_norm = pl.pallas_call(
    _norm_body,
    grid=(N // _BN,),
    in_specs=[pl.BlockSpec((_BN, 2 * D), lambda i: (i, 0)),
              pl.BlockSpec((_BN, 2 * D), lambda i: (i, 0)),
              pl.BlockSpec((_BN, D), lambda i: (i, 0)),
              pl.BlockSpec((8, D), lambda i: (0, 0)),
              pl.BlockSpec((1, D), lambda i: (0, 0)),
              pl.BlockSpec((1, D), lambda i: (0, 0))],
    out_specs=pl.BlockSpec((_BN, D), lambda i: (i, 0)),
    out_shape=jax.ShapeDtypeStruct((N, D), _F32),
)


def _node_init_body(x_ref, bt_ref, chg_ref, wch_ref, bch_ref, wa_ref, ba_ref,
                    out_ref):
    ch = chg_ref[:, 0:1] * wch_ref[...] + bch_ref[...]
    chw = jnp.dot(ch, wa_ref[128:128 + 16, :],
                  preferred_element_type=_F32)
    ids = lax.broadcasted_iota(jnp.int32, (1, G), 1)
    oh = (bt_ref[...] == ids).astype(_F32)
    y = (jnp.dot(x_ref[...], wa_ref[0:128, :], preferred_element_type=_F32)
         + jnp.dot(oh, chw, preferred_element_type=_F32) + ba_ref[...])
    out_ref[...] = y


_node_init = pl.pallas_call(
    _node_init_body,
    grid=(N // _BN,),
    in_specs=[pl.BlockSpec((_BN, 128), lambda i: (i, 0)),
              pl.BlockSpec((_BN, 1), lambda i: (i, 0)),
              pl.BlockSpec((G, 128), lambda i: (0, 0)),
              pl.BlockSpec((1, G), lambda i: (0, 0)),
              pl.BlockSpec((1, G), lambda i: (0, 0)),
              pl.BlockSpec((144, D), lambda i: (0, 0)),
              pl.BlockSpec((1, D), lambda i: (0, 0))],
    out_specs=pl.BlockSpec((_BN, D), lambda i: (i, 0)),
    out_shape=jax.ShapeDtypeStruct((N, D), _F32),
)


def _edge_init_body(ea_ref, wb_ref, bb_ref, out_ref):
    ea0 = (jnp.dot(ea_ref[...], wb_ref[...],
                   preferred_element_type=_F32) + bb_ref[...])
    out_ref[...] = jnp.concatenate([jnp.zeros((_BE, D), _F32), ea0], axis=1)


_edge_init = pl.pallas_call(
    _edge_init_body,
    grid=(E2 // _BE,),
    in_specs=[pl.BlockSpec((_BE, 16), lambda i: (i, 0)),
              pl.BlockSpec((16, D), lambda i: (0, 0)),
              pl.BlockSpec((1, D), lambda i: (0, 0))],
    out_specs=pl.BlockSpec((_BE, 2 * D), lambda i: (i, 0)),
    out_shape=jax.ShapeDtypeStruct((E2, 2 * D), _F32),
)


def _pool_body(x_ref, bt_ref, out_ref):
    i = pl.program_id(0)

    @pl.when(i == 0)
    def _():
        out_ref[...] = jnp.zeros_like(out_ref)

    ids = lax.broadcasted_iota(jnp.int32, (1, G), 1)
    oh = (bt_ref[...] == ids).astype(_F32)
    ones = jnp.ones((_BN, 1), _F32)
    zeros = jnp.zeros((_BN, 128 - D - 1), _F32)
    aug = jnp.concatenate([x_ref[...], ones, zeros], axis=1)
    out_ref[...] += jnp.dot(oh.T, aug, preferred_element_type=_F32)


_pool = pl.pallas_call(
    _pool_body,
    grid=(N // _BN,),
    in_specs=[pl.BlockSpec((_BN, D), lambda i: (i, 0)),
              pl.BlockSpec((_BN, 1), lambda i: (i, 0))],
    out_specs=pl.BlockSpec((G, 128), lambda i: (0, 0)),
    out_shape=jax.ShapeDtypeStruct((G, 128), _F32),
)


def _head_body(po_ref, wp1_ref, bp1_ref, wp2_ref, bp2_ref, wp3_ref, bp3_ref,
               out_ref):
    po = po_ref[...]
    gr = po[:, :D] / jnp.maximum(po[:, D:D + 1], 1.0)
    h = _sp(jnp.dot(gr, wp1_ref[...], preferred_element_type=_F32)
            + bp1_ref[...])
    h = _sp(jnp.dot(h, wp2_ref[...], preferred_element_type=_F32)
            + bp2_ref[...])
    out_ref[...] = (jnp.dot(h, wp3_ref[...], preferred_element_type=_F32)
                    + bp3_ref[...])


_head = pl.pallas_call(
    _head_body,
    grid=(1,),
    in_specs=[pl.BlockSpec((G, 128), lambda i: (0, 0)),
              pl.BlockSpec((D, 128), lambda i: (0, 0)),
              pl.BlockSpec((1, 128), lambda i: (0, 0)),
              pl.BlockSpec((128, 128), lambda i: (0, 0)),
              pl.BlockSpec((1, 128), lambda i: (0, 0)),
              pl.BlockSpec((128, 8), lambda i: (0, 0)),
              pl.BlockSpec((1, 8), lambda i: (0, 0))],
    out_specs=pl.BlockSpec((G, 8), lambda i: (0, 0)),
    out_shape=jax.ShapeDtypeStruct((G, 8), _F32),
)


def kernel(x, edge_attr, charge, params, edge_index, batch):
    p = params
    rowA, rowB = edge_index[0, :E2], edge_index[0, E2:]
    colA, colB = edge_index[1, :E2], edge_index[1, E2:]
    batch2 = batch[:, None]
    charge_b = jnp.broadcast_to(charge[:, None], (G, 128))
    zeros32 = jnp.zeros((_ZR, _DH), _F32)

    xcur = _node_init(x, batch2, charge_b, p['Wch'], p['bch'][None, :],
                      p['Wa'], p['ba'][None, :])
    pkA = _edge_init(edge_attr[:E2], p['Wb'], p['bb'][None, :])
    pkB = _edge_init(edge_attr[E2:], p['Wb'], p['bb'][None, :])

    for i in range(L):
        w1 = jnp.zeros((3 * D, 2 * D), _BF)
        w1 = w1.at[:, :D].set(p[f'We1_{i}'].astype(_BF))
        w1 = w1.at[0:D, D:].set(p[f'Wn1_{i}'][0:D].astype(_BF))
        b1 = jnp.concatenate([p[f'be1_{i}'], p[f'bn1_{i}']])[None, :]
        ew = (w1, b1, p[f'We2_{i}'].astype(_BF), p[f'be2_{i}'][None, :],
              p[f'Wn1_{i}'][D:2 * D].astype(_BF),
              p[f'Wn2_{i}'].astype(_BF), p[f'bn2_{i}'][None, :])
        (xrcA,) = _sc_gather(xcur, rowA, colA)
        (xrcB,) = _sc_gather(xcur, rowB, colB)
        pkA = _edge_mlp(xrcA, pkA, *ew)
        pkB = _edge_mlp(xrcB, pkB, *ew)
        (xnA,) = _sc_scatter(pkA, colA, zeros32)
        (xnB,) = _sc_scatter(pkB, colB, zeros32)
        st = _stats(xnA, xnB)
        xcur = _norm(xnA, xnB, xcur, st,
                     p[f'g_{i}'][None, :], p[f'bt_{i}'][None, :])

    po = _pool(xcur, batch2)
    out = _head(po, p['Wp1'], p['bp1'][None, :], p['Wp2'], p['bp2'][None, :],
                jnp.pad(p['Wp3'], ((0, 0), (0, 7))),
                jnp.pad(p['bp3'][None, :], ((0, 0), (0, 7))))
    return out[:, 0]


# speedup vs baseline: 1.7343x; 1.0648x over previous
import functools

import jax
import jax.numpy as jnp
from jax import lax
from jax.experimental import pallas as pl
from jax.experimental.pallas import tpu as pltpu
from jax.experimental.pallas import tpu_sc as plsc

N = 50000
E = 800000
D = 64
G = 16
L = 3

_NC = 2
_NS = 16
_NW = _NC * _NS

_F32 = jnp.float32

E2 = E // 2
_GCH = 128
_GW = 12544
_GK0 = _GW // _GCH
_GWL = E2 - _GW * (_NW - 1)
_GKL = _GWL // _GCH


def _sc_gather_body(x_hbm, row_hbm, col_hbm, xrc_hbm,
                    idx_all, buf0, buf1, sem0, sem1):
    c = lax.axis_index("c")
    s = lax.axis_index("s")
    wid = c * _NS + s
    base = wid * _GW
    nk = jnp.where(wid < _NW - 1, _GK0, _GKL)

    def do_phase(src_idx_hbm, cbase):
        @pl.when(wid < _NW - 1)
        def _():
            pltpu.sync_copy(src_idx_hbm.at[pl.ds(base, _GW)], idx_all)

        @pl.when(wid == _NW - 1)
        def _():
            pltpu.sync_copy(src_idx_hbm.at[pl.ds(base, _GWL)],
                            idx_all.at[pl.ds(0, _GWL)])

        def fire(k, buf, sem):
            pltpu.async_copy(x_hbm.at[idx_all.at[pl.ds(k * _GCH, _GCH)]],
                             buf, sem)

        def wait(k, buf, sem):
            pltpu.make_async_copy(x_hbm.at[idx_all.at[pl.ds(k * _GCH, _GCH)]],
                                  buf, sem).wait()

        fire(0, buf0, sem0)

        def body(k, _):
            eb = base + k * _GCH

            @pl.when(k % 2 == 0)
            def _():
                @pl.when(k + 1 < nk)
                def _():
                    fire(k + 1, buf1, sem1)
                wait(k, buf0, sem0)
                pltpu.sync_copy(buf0, xrc_hbm.at[pl.ds(eb, _GCH),
                                                 pl.ds(cbase, D)])

            @pl.when(k % 2 == 1)
            def _():
                @pl.when(k + 1 < nk)
                def _():
                    fire(k + 1, buf0, sem0)
                wait(k, buf1, sem1)
                pltpu.sync_copy(buf1, xrc_hbm.at[pl.ds(eb, _GCH),
                                                 pl.ds(cbase, D)])
            return 0

        lax.fori_loop(0, nk, body, 0)

    do_phase(row_hbm, 0)
    do_phase(col_hbm, D)


_sc_gather = pl.kernel(
    _sc_gather_body,
    out_type=[jax.ShapeDtypeStruct((E2, 2 * D), _F32)],
    mesh=plsc.VectorSubcoreMesh(core_axis_name="c", subcore_axis_name="s"),
    scratch_types=[pltpu.VMEM((_GW,), jnp.int32),
                   pltpu.VMEM((_GCH, D), _F32),
                   pltpu.VMEM((_GCH, D), _F32),
                   pltpu.SemaphoreType.DMA,
                   pltpu.SemaphoreType.DMA],
    compiler_params=pltpu.CompilerParams(use_tc_tiling_on_sc=False),
)

_DH = D // 2
_EPT = E2 // _NS
_SCH = 128
_SFULL = _EPT // _SCH
_STAIL = _EPT - _SFULL * _SCH
_ZR = 3200
_ZR_LAST = N - (_NS - 1) * _ZR


def _sc_scatter_body(pk_hbm, col_hbm, zero_hbm, xn_hbm,
                     acc, cidx0, cidx1, mbuf0, mbuf1, cidx_t, mbuf_t,
                     csem0, csem1, msem0, msem1):
    c = lax.axis_index("c")
    s = lax.axis_index("s")

    @pl.when(s < _NS - 1)
    def _():
        pltpu.sync_copy(zero_hbm, acc.at[pl.ds(s * _ZR, _ZR)])

    @pl.when(s == _NS - 1)
    def _():
        pltpu.sync_copy(zero_hbm.at[pl.ds(0, _ZR_LAST)],
                        acc.at[pl.ds((_NS - 1) * _ZR, _ZR_LAST)])

    plsc.subcore_barrier()

    def scat(cb):
        ebase = s * _EPT

        def fire(eb, cidx, mbuf, csem, msem):
            pltpu.async_copy(col_hbm.at[pl.ds(eb, _SCH)], cidx, csem)
            pltpu.async_copy(pk_hbm.at[pl.ds(eb, _SCH), pl.ds(cb, _DH)],
                             mbuf, msem)

        def wait(eb, cidx, mbuf, csem, msem):
            pltpu.make_async_copy(col_hbm.at[pl.ds(eb, _SCH)],
                                  cidx, csem).wait()
            pltpu.make_async_copy(pk_hbm.at[pl.ds(eb, _SCH), pl.ds(cb, _DH)],
                                  mbuf, msem).wait()

        fire(ebase, cidx0, mbuf0, csem0, msem0)

        def body(j, _):
            eb = ebase + j * _SCH
            eb_n = eb + _SCH

            @pl.when(j % 2 == 0)
            def _():
                wait(eb, cidx0, mbuf0, csem0, msem0)

                @pl.when(j + 1 < _SFULL)
                def _():
                    fire(eb_n, cidx1, mbuf1, csem1, msem1)
                pltpu.sync_copy(mbuf0, acc.at[cidx0], add=True)

            @pl.when(j % 2 == 1)
            def _():
                wait(eb, cidx1, mbuf1, csem1, msem1)

                @pl.when(j + 1 < _SFULL)
                def _():
                    fire(eb_n, cidx0, mbuf0, csem0, msem0)
                pltpu.sync_copy(mbuf1, acc.at[cidx1], add=True)
            return 0

        lax.fori_loop(0, _SFULL, body, 0)
        eb = ebase + _SFULL * _SCH
        pltpu.sync_copy(col_hbm.at[pl.ds(eb, _STAIL)], cidx_t)
        pltpu.sync_copy(pk_hbm.at[pl.ds(eb, _STAIL), pl.ds(cb, _DH)], mbuf_t)
        pltpu.sync_copy(mbuf_t, acc.at[cidx_t], add=True)

    @pl.when(c == 0)
    def _():
        scat(0)

    @pl.when(c == 1)
    def _():
        scat(_DH)

    plsc.subcore_barrier()

    def wout(cb):
        @pl.when(s < _NS - 1)
        def _():
            pltpu.sync_copy(acc.at[pl.ds(s * _ZR, _ZR)],
                            xn_hbm.at[pl.ds(s * _ZR, _ZR), pl.ds(cb, _DH)])

        @pl.when(s == _NS - 1)
        def _():
            pltpu.sync_copy(acc.at[pl.ds((_NS - 1) * _ZR, _ZR_LAST)],
                            xn_hbm.at[pl.ds((_NS - 1) * _ZR, _ZR_LAST),
                                      pl.ds(cb, _DH)])

    @pl.when(c == 0)
    def _():
        wout(0)

    @pl.when(c == 1)
    def _():
        wout(_DH)


_sc_scatter = pl.kernel(
    _sc_scatter_body,
    out_type=[jax.ShapeDtypeStruct((N, 2 * D), _F32)],
    mesh=plsc.VectorSubcoreMesh(core_axis_name="c", subcore_axis_name="s"),
    scratch_types=[pltpu.VMEM_SHARED((N, _DH), _F32),
                   pltpu.VMEM((_SCH,), jnp.int32),
                   pltpu.VMEM((_SCH,), jnp.int32),
                   pltpu.VMEM((_SCH, _DH), _F32),
                   pltpu.VMEM((_SCH, _DH), _F32),
                   pltpu.VMEM((_STAIL,), jnp.int32),
                   pltpu.VMEM((_STAIL, _DH), _F32),
                   pltpu.SemaphoreType.DMA,
                   pltpu.SemaphoreType.DMA,
                   pltpu.SemaphoreType.DMA,
                   pltpu.SemaphoreType.DMA],
    compiler_params=pltpu.CompilerParams(use_tc_tiling_on_sc=False),
)


def _sp(v):
    return jnp.maximum(v, 0.0) + jnp.log1p(jnp.exp(-jnp.abs(v)))


_BE = 8000
_BN = 5000
_BF = jnp.bfloat16


def _make_edge_mlp(first):
    def body(xrc_ref, pk_ref, w1_ref, b1_ref, we2_ref, be2_ref,
             wn1b_ref, wn2_ref, bn2_ref, pk_out):
        ea_prev = pk_ref[...] if first else pk_ref[:, D:]
        lhs = jnp.concatenate([xrc_ref[...], ea_prev], axis=1).astype(_BF)
        t = (jnp.dot(lhs, w1_ref[...], preferred_element_type=_F32)
             + b1_ref[...])
        h = _sp(t[:, :D])
        ea2 = (jnp.dot(h.astype(_BF), we2_ref[...],
                       preferred_element_type=_F32) + be2_ref[...])
        m = _sp(t[:, D:]
                + jnp.dot(ea2.astype(_BF), wn1b_ref[...],
                          preferred_element_type=_F32))
        msg = (jnp.dot(m.astype(_BF), wn2_ref[...],
                       preferred_element_type=_F32) + bn2_ref[...])
        pk_out[...] = jnp.concatenate([msg, ea2], axis=1)

    pk_w = D if first else 2 * D
    return pl.pallas_call(
        body,
        grid=(E2 // _BE,),
        in_specs=[
            pl.BlockSpec((_BE, 2 * D), lambda i: (i, 0)),
            pl.BlockSpec((_BE, pk_w), lambda i: (i, 0)),
            pl.BlockSpec((3 * D, 2 * D), lambda i: (0, 0)),
            pl.BlockSpec((1, 2 * D), lambda i: (0, 0)),
            pl.BlockSpec((D, D), lambda i: (0, 0)),
            pl.BlockSpec((1, D), lambda i: (0, 0)),
            pl.BlockSpec((D, D), lambda i: (0, 0)),
            pl.BlockSpec((D, D), lambda i: (0, 0)),
            pl.BlockSpec((1, D), lambda i: (0, 0)),
        ],
        out_specs=pl.BlockSpec((_BE, 2 * D), lambda i: (i, 0)),
        out_shape=jax.ShapeDtypeStruct((E2, 2 * D), _F32),
        input_output_aliases={} if first else {1: 0},
    )


_edge_mlp_first = _make_edge_mlp(True)
_edge_mlp_next = _make_edge_mlp(False)


def _stats_body(a_ref, b_ref, out_ref):
    i = pl.program_id(0)

    @pl.when(i == 0)
    def _():
        out_ref[...] = jnp.zeros_like(out_ref)

    xn = a_ref[:, :D] + b_ref[:, :D]
    out_ref[0:1, :] += jnp.sum(xn, axis=0)[None, :]
    out_ref[1:2, :] += jnp.sum(xn * xn, axis=0)[None, :]


_stats = pl.pallas_call(
    _stats_body,
    grid=(N // _BN,),
    in_specs=[pl.BlockSpec((_BN, 2 * D), lambda i: (i, 0)),
              pl.BlockSpec((_BN, 2 * D), lambda i: (i, 0))],
    out_specs=pl.BlockSpec((8, D), lambda i: (0, 0)),
    out_shape=jax.ShapeDtypeStruct((8, D), _F32),
)


def _norm_body(a_ref, b_ref, xp_ref, st_ref, g_ref, bt_ref, out_ref):
    xn = a_ref[:, :D] + b_ref[:, :D]
    mu = st_ref[0:1, :] / N
    var = st_ref[1:2, :] / N - mu * mu
    inv = lax.rsqrt(var + 1e-5)
    y = g_ref[...] * (xn - mu) * inv + bt_ref[...]
    out_ref[...] = _sp(y) + xp_ref[...]


_norm = pl.pallas_call(
    _norm_body,
    grid=(N // _BN,),
    in_specs=[pl.BlockSpec((_BN, 2 * D), lambda i: (i, 0)),
              pl.BlockSpec((_BN, 2 * D), lambda i: (i, 0)),
              pl.BlockSpec((_BN, D), lambda i: (i, 0)),
              pl.BlockSpec((8, D), lambda i: (0, 0)),
              pl.BlockSpec((1, D), lambda i: (0, 0)),
              pl.BlockSpec((1, D), lambda i: (0, 0))],
    out_specs=pl.BlockSpec((_BN, D), lambda i: (i, 0)),
    out_shape=jax.ShapeDtypeStruct((N, D), _F32),
)


def _node_init_body(x_ref, bt_ref, chg_ref, wch_ref, bch_ref, wa_ref, ba_ref,
                    out_ref):
    ch = chg_ref[:, 0:1] * wch_ref[...] + bch_ref[...]
    chw = jnp.dot(ch, wa_ref[128:128 + 16, :],
                  preferred_element_type=_F32)
    ids = lax.broadcasted_iota(jnp.int32, (1, G), 1)
    oh = (bt_ref[...] == ids).astype(_F32)
    y = (jnp.dot(x_ref[...], wa_ref[0:128, :], preferred_element_type=_F32)
         + jnp.dot(oh, chw, preferred_element_type=_F32) + ba_ref[...])
    out_ref[...] = y


_node_init = pl.pallas_call(
    _node_init_body,
    grid=(N // _BN,),
    in_specs=[pl.BlockSpec((_BN, 128), lambda i: (i, 0)),
              pl.BlockSpec((_BN, 1), lambda i: (i, 0)),
              pl.BlockSpec((G, 128), lambda i: (0, 0)),
              pl.BlockSpec((1, G), lambda i: (0, 0)),
              pl.BlockSpec((1, G), lambda i: (0, 0)),
              pl.BlockSpec((144, D), lambda i: (0, 0)),
              pl.BlockSpec((1, D), lambda i: (0, 0))],
    out_specs=pl.BlockSpec((_BN, D), lambda i: (i, 0)),
    out_shape=jax.ShapeDtypeStruct((N, D), _F32),
)


def _edge_init_body(ea_ref, wb_ref, bb_ref, out_ref):
    out_ref[...] = (jnp.dot(ea_ref[...], wb_ref[...],
                            preferred_element_type=_F32) + bb_ref[...])


_edge_init = pl.pallas_call(
    _edge_init_body,
    grid=(E2 // _BE,),
    in_specs=[pl.BlockSpec((_BE, 16), lambda i: (i, 0)),
              pl.BlockSpec((16, D), lambda i: (0, 0)),
              pl.BlockSpec((1, D), lambda i: (0, 0))],
    out_specs=pl.BlockSpec((_BE, D), lambda i: (i, 0)),
    out_shape=jax.ShapeDtypeStruct((E2, D), _F32),
)


def _pool_body(x_ref, bt_ref, out_ref):
    i = pl.program_id(0)

    @pl.when(i == 0)
    def _():
        out_ref[...] = jnp.zeros_like(out_ref)

    ids = lax.broadcasted_iota(jnp.int32, (1, G), 1)
    oh = (bt_ref[...] == ids).astype(_F32)
    ones = jnp.ones((_BN, 1), _F32)
    zeros = jnp.zeros((_BN, 128 - D - 1), _F32)
    aug = jnp.concatenate([x_ref[...], ones, zeros], axis=1)
    out_ref[...] += jnp.dot(oh.T, aug, preferred_element_type=_F32)


_pool = pl.pallas_call(
    _pool_body,
    grid=(N // _BN,),
    in_specs=[pl.BlockSpec((_BN, D), lambda i: (i, 0)),
              pl.BlockSpec((_BN, 1), lambda i: (i, 0))],
    out_specs=pl.BlockSpec((G, 128), lambda i: (0, 0)),
    out_shape=jax.ShapeDtypeStruct((G, 128), _F32),
)


def _head_body(po_ref, wp1_ref, bp1_ref, wp2_ref, bp2_ref, wp3_ref, bp3_ref,
               out_ref):
    po = po_ref[...]
    gr = po[:, :D] / jnp.maximum(po[:, D:D + 1], 1.0)
    h = _sp(jnp.dot(gr, wp1_ref[...], preferred_element_type=_F32)
            + bp1_ref[...])
    h = _sp(jnp.dot(h, wp2_ref[...], preferred_element_type=_F32)
            + bp2_ref[...])
    out_ref[...] = (jnp.dot(h, wp3_ref[...], preferred_element_type=_F32)
                    + bp3_ref[...])


_head = pl.pallas_call(
    _head_body,
    grid=(1,),
    in_specs=[pl.BlockSpec((G, 128), lambda i: (0, 0)),
              pl.BlockSpec((D, 128), lambda i: (0, 0)),
              pl.BlockSpec((1, 128), lambda i: (0, 0)),
              pl.BlockSpec((128, 128), lambda i: (0, 0)),
              pl.BlockSpec((1, 128), lambda i: (0, 0)),
              pl.BlockSpec((128, 8), lambda i: (0, 0)),
              pl.BlockSpec((1, 8), lambda i: (0, 0))],
    out_specs=pl.BlockSpec((G, 8), lambda i: (0, 0)),
    out_shape=jax.ShapeDtypeStruct((G, 8), _F32),
)


def kernel(x, edge_attr, charge, params, edge_index, batch):
    p = params
    rowA, rowB = edge_index[0, :E2], edge_index[0, E2:]
    colA, colB = edge_index[1, :E2], edge_index[1, E2:]
    batch2 = batch[:, None]
    charge_b = jnp.broadcast_to(charge[:, None], (G, 128))
    zeros32 = jnp.zeros((_ZR, _DH), _F32)

    xcur = _node_init(x, batch2, charge_b, p['Wch'], p['bch'][None, :],
                      p['Wa'], p['ba'][None, :])
    pkA = _edge_init(edge_attr[:E2], p['Wb'], p['bb'][None, :])
    pkB = _edge_init(edge_attr[E2:], p['Wb'], p['bb'][None, :])

    for i in range(L):
        w1 = jnp.zeros((3 * D, 2 * D), _BF)
        w1 = w1.at[:, :D].set(p[f'We1_{i}'].astype(_BF))
        w1 = w1.at[0:D, D:].set(p[f'Wn1_{i}'][0:D].astype(_BF))
        b1 = jnp.concatenate([p[f'be1_{i}'], p[f'bn1_{i}']])[None, :]
        ew = (w1, b1, p[f'We2_{i}'].astype(_BF), p[f'be2_{i}'][None, :],
              p[f'Wn1_{i}'][D:2 * D].astype(_BF),
              p[f'Wn2_{i}'].astype(_BF), p[f'bn2_{i}'][None, :])
        em = _edge_mlp_first if i == 0 else _edge_mlp_next
        (xrcA,) = _sc_gather(xcur, rowA, colA)
        (xrcB,) = _sc_gather(xcur, rowB, colB)
        pkA = em(xrcA, pkA, *ew)
        pkB = em(xrcB, pkB, *ew)
        (xnA,) = _sc_scatter(pkA, colA, zeros32)
        (xnB,) = _sc_scatter(pkB, colB, zeros32)
        st = _stats(xnA, xnB)
        xcur = _norm(xnA, xnB, xcur, st,
                     p[f'g_{i}'][None, :], p[f'bt_{i}'][None, :])

    po = _pool(xcur, batch2)
    out = _head(po, p['Wp1'], p['bp1'][None, :], p['Wp2'], p['bp2'][None, :],
                jnp.pad(p['Wp3'], ((0, 0), (0, 7))),
                jnp.pad(p['bp3'][None, :], ((0, 0), (0, 7))))
    return out[:, 0]


# speedup vs baseline: 1.8733x; 1.0801x over previous
import functools

import jax
import jax.numpy as jnp
from jax import lax
from jax.experimental import pallas as pl
from jax.experimental.pallas import tpu as pltpu
from jax.experimental.pallas import tpu_sc as plsc

N = 50000
E = 800000
D = 64
G = 16
L = 3

_NC = 2
_NS = 16
_NW = _NC * _NS

_F32 = jnp.float32

E2 = E // 2
_GCH = 128
_GW = 12544
_GK0 = _GW // _GCH
_GWL = E2 - _GW * (_NW - 1)
_GKL = _GWL // _GCH


def _sc_gather_body(x_hbm, row_hbm, col_hbm, xrc_hbm,
                    idx_all, buf0, buf1, sem0, sem1):
    c = lax.axis_index("c")
    s = lax.axis_index("s")
    wid = c * _NS + s
    base = wid * _GW
    nk = jnp.where(wid < _NW - 1, _GK0, _GKL)

    def do_phase(src_idx_hbm, cbase):
        @pl.when(wid < _NW - 1)
        def _():
            pltpu.sync_copy(src_idx_hbm.at[pl.ds(base, _GW)], idx_all)

        @pl.when(wid == _NW - 1)
        def _():
            pltpu.sync_copy(src_idx_hbm.at[pl.ds(base, _GWL)],
                            idx_all.at[pl.ds(0, _GWL)])

        def fire(k, buf, sem):
            pltpu.async_copy(x_hbm.at[idx_all.at[pl.ds(k * _GCH, _GCH)]],
                             buf, sem)

        def wait(k, buf, sem):
            pltpu.make_async_copy(x_hbm.at[idx_all.at[pl.ds(k * _GCH, _GCH)]],
                                  buf, sem).wait()

        fire(0, buf0, sem0)

        def body(k, _):
            eb = base + k * _GCH

            @pl.when(k % 2 == 0)
            def _():
                @pl.when(k + 1 < nk)
                def _():
                    fire(k + 1, buf1, sem1)
                wait(k, buf0, sem0)
                pltpu.sync_copy(buf0, xrc_hbm.at[pl.ds(eb, _GCH),
                                                 pl.ds(cbase, D)])

            @pl.when(k % 2 == 1)
            def _():
                @pl.when(k + 1 < nk)
                def _():
                    fire(k + 1, buf0, sem0)
                wait(k, buf1, sem1)
                pltpu.sync_copy(buf1, xrc_hbm.at[pl.ds(eb, _GCH),
                                                 pl.ds(cbase, D)])
            return 0

        lax.fori_loop(0, nk, body, 0)

    do_phase(row_hbm, 0)
    do_phase(col_hbm, D)


_sc_gather = pl.kernel(
    _sc_gather_body,
    out_type=[jax.ShapeDtypeStruct((E2, 2 * D), _F32)],
    mesh=plsc.VectorSubcoreMesh(core_axis_name="c", subcore_axis_name="s"),
    scratch_types=[pltpu.VMEM((_GW,), jnp.int32),
                   pltpu.VMEM((_GCH, D), _F32),
                   pltpu.VMEM((_GCH, D), _F32),
                   pltpu.SemaphoreType.DMA,
                   pltpu.SemaphoreType.DMA],
    compiler_params=pltpu.CompilerParams(use_tc_tiling_on_sc=False),
)

_DH = D // 2
_EPT = E2 // _NS
_SCH = 128
_SFULL = _EPT // _SCH
_STAIL = _EPT - _SFULL * _SCH
_ZR = 3200
_ZR_LAST = N - (_NS - 1) * _ZR


def _sc_scatter_body(pk_hbm, col_hbm, zero_hbm, xn_hbm,
                     acc, cidx0, cidx1, mbuf0, mbuf1, cidx_t, mbuf_t,
                     csem0, csem1, msem0, msem1):
    c = lax.axis_index("c")
    s = lax.axis_index("s")

    @pl.when(s < _NS - 1)
    def _():
        pltpu.sync_copy(zero_hbm, acc.at[pl.ds(s * _ZR, _ZR)])

    @pl.when(s == _NS - 1)
    def _():
        pltpu.sync_copy(zero_hbm.at[pl.ds(0, _ZR_LAST)],
                        acc.at[pl.ds((_NS - 1) * _ZR, _ZR_LAST)])

    plsc.subcore_barrier()

    def scat(cb):
        ebase = s * _EPT

        def fire(eb, cidx, mbuf, csem, msem):
            pltpu.async_copy(col_hbm.at[pl.ds(eb, _SCH)], cidx, csem)
            pltpu.async_copy(pk_hbm.at[pl.ds(eb, _SCH), pl.ds(cb, _DH)],
                             mbuf, msem)

        def wait(eb, cidx, mbuf, csem, msem):
            pltpu.make_async_copy(col_hbm.at[pl.ds(eb, _SCH)],
                                  cidx, csem).wait()
            pltpu.make_async_copy(pk_hbm.at[pl.ds(eb, _SCH), pl.ds(cb, _DH)],
                                  mbuf, msem).wait()

        fire(ebase, cidx0, mbuf0, csem0, msem0)

        def body(j, _):
            eb = ebase + j * _SCH
            eb_n = eb + _SCH

            @pl.when(j % 2 == 0)
            def _():
                wait(eb, cidx0, mbuf0, csem0, msem0)

                @pl.when(j + 1 < _SFULL)
                def _():
                    fire(eb_n, cidx1, mbuf1, csem1, msem1)
                pltpu.sync_copy(mbuf0, acc.at[cidx0], add=True)

            @pl.when(j % 2 == 1)
            def _():
                wait(eb, cidx1, mbuf1, csem1, msem1)

                @pl.when(j + 1 < _SFULL)
                def _():
                    fire(eb_n, cidx0, mbuf0, csem0, msem0)
                pltpu.sync_copy(mbuf1, acc.at[cidx1], add=True)
            return 0

        lax.fori_loop(0, _SFULL, body, 0)
        eb = ebase + _SFULL * _SCH
        pltpu.sync_copy(col_hbm.at[pl.ds(eb, _STAIL)], cidx_t)
        pltpu.sync_copy(pk_hbm.at[pl.ds(eb, _STAIL), pl.ds(cb, _DH)], mbuf_t)
        pltpu.sync_copy(mbuf_t, acc.at[cidx_t], add=True)

    @pl.when(c == 0)
    def _():
        scat(0)

    @pl.when(c == 1)
    def _():
        scat(_DH)

    plsc.subcore_barrier()

    def wout(cb):
        @pl.when(s < _NS - 1)
        def _():
            pltpu.sync_copy(acc.at[pl.ds(s * _ZR, _ZR)],
                            xn_hbm.at[pl.ds(s * _ZR, _ZR), pl.ds(cb, _DH)])

        @pl.when(s == _NS - 1)
        def _():
            pltpu.sync_copy(acc.at[pl.ds((_NS - 1) * _ZR, _ZR_LAST)],
                            xn_hbm.at[pl.ds((_NS - 1) * _ZR, _ZR_LAST),
                                      pl.ds(cb, _DH)])

    @pl.when(c == 0)
    def _():
        wout(0)

    @pl.when(c == 1)
    def _():
        wout(_DH)


_sc_scatter = pl.kernel(
    _sc_scatter_body,
    out_type=[jax.ShapeDtypeStruct((N, 2 * D), _F32)],
    mesh=plsc.VectorSubcoreMesh(core_axis_name="c", subcore_axis_name="s"),
    scratch_types=[pltpu.VMEM_SHARED((N, _DH), _F32),
                   pltpu.VMEM((_SCH,), jnp.int32),
                   pltpu.VMEM((_SCH,), jnp.int32),
                   pltpu.VMEM((_SCH, _DH), _F32),
                   pltpu.VMEM((_SCH, _DH), _F32),
                   pltpu.VMEM((_STAIL,), jnp.int32),
                   pltpu.VMEM((_STAIL, _DH), _F32),
                   pltpu.SemaphoreType.DMA,
                   pltpu.SemaphoreType.DMA,
                   pltpu.SemaphoreType.DMA,
                   pltpu.SemaphoreType.DMA],
    compiler_params=pltpu.CompilerParams(use_tc_tiling_on_sc=False),
)


def _sp(v):
    return jnp.maximum(v, 0.0) + jnp.log1p(jnp.exp(-jnp.abs(v)))


_BE = 8000
_BN = 5000
_BF = jnp.bfloat16


def _make_edge_mlp(first):
    def body(xrc_ref, pk_ref, w1_ref, b1_ref, we2_ref, be2_ref,
             wn1b_ref, wn2_ref, bn2_ref, wb_ref, bb_ref, pk_out):
        if first:
            ea_prev = (jnp.dot(pk_ref[...], wb_ref[...],
                               preferred_element_type=_F32) + bb_ref[...])
        else:
            ea_prev = pk_ref[:, D:]
        lhs = jnp.concatenate([xrc_ref[...], ea_prev], axis=1).astype(_BF)
        t = (jnp.dot(lhs, w1_ref[...], preferred_element_type=_F32)
             + b1_ref[...])
        h = _sp(t[:, :D])
        ea2 = (jnp.dot(h.astype(_BF), we2_ref[...],
                       preferred_element_type=_F32) + be2_ref[...])
        m = _sp(t[:, D:]
                + jnp.dot(ea2.astype(_BF), wn1b_ref[...],
                          preferred_element_type=_F32))
        msg = (jnp.dot(m.astype(_BF), wn2_ref[...],
                       preferred_element_type=_F32) + bn2_ref[...])
        pk_out[...] = jnp.concatenate([msg, ea2], axis=1)

    pk_w = 16 if first else 2 * D
    return pl.pallas_call(
        body,
        grid=(E2 // _BE,),
        in_specs=[
            pl.BlockSpec((_BE, 2 * D), lambda i: (i, 0)),
            pl.BlockSpec((_BE, pk_w), lambda i: (i, 0)),
            pl.BlockSpec((3 * D, 2 * D), lambda i: (0, 0)),
            pl.BlockSpec((1, 2 * D), lambda i: (0, 0)),
            pl.BlockSpec((D, D), lambda i: (0, 0)),
            pl.BlockSpec((1, D), lambda i: (0, 0)),
            pl.BlockSpec((D, D), lambda i: (0, 0)),
            pl.BlockSpec((D, D), lambda i: (0, 0)),
            pl.BlockSpec((1, D), lambda i: (0, 0)),
            pl.BlockSpec((16, D), lambda i: (0, 0)),
            pl.BlockSpec((1, D), lambda i: (0, 0)),
        ],
        out_specs=pl.BlockSpec((_BE, 2 * D), lambda i: (i, 0)),
        out_shape=jax.ShapeDtypeStruct((E2, 2 * D), _F32),
        input_output_aliases={} if first else {1: 0},
    )


_edge_mlp_first = _make_edge_mlp(True)
_edge_mlp_next = _make_edge_mlp(False)


def _stats_body(a_ref, b_ref, out_ref):
    i = pl.program_id(0)

    @pl.when(i == 0)
    def _():
        out_ref[...] = jnp.zeros_like(out_ref)

    xn = a_ref[:, :D] + b_ref[:, :D]
    out_ref[0:1, :] += jnp.sum(xn, axis=0)[None, :]
    out_ref[1:2, :] += jnp.sum(xn * xn, axis=0)[None, :]


_stats = pl.pallas_call(
    _stats_body,
    grid=(N // _BN,),
    in_specs=[pl.BlockSpec((_BN, 2 * D), lambda i: (i, 0)),
              pl.BlockSpec((_BN, 2 * D), lambda i: (i, 0))],
    out_specs=pl.BlockSpec((8, D), lambda i: (0, 0)),
    out_shape=jax.ShapeDtypeStruct((8, D), _F32),
)


def _norm_body(a_ref, b_ref, xp_ref, st_ref, g_ref, bt_ref, out_ref):
    xn = a_ref[:, :D] + b_ref[:, :D]
    mu = st_ref[0:1, :] / N
    var = st_ref[1:2, :] / N - mu * mu
    inv = lax.rsqrt(var + 1e-5)
    y = g_ref[...] * (xn - mu) * inv + bt_ref[...]
    out_ref[...] = _sp(y) + xp_ref[...]


_norm = pl.pallas_call(
    _norm_body,
    grid=(N // _BN,),
    in_specs=[pl.BlockSpec((_BN, 2 * D), lambda i: (i, 0)),
              pl.BlockSpec((_BN, 2 * D), lambda i: (i, 0)),
              pl.BlockSpec((_BN, D), lambda i: (i, 0)),
              pl.BlockSpec((8, D), lambda i: (0, 0)),
              pl.BlockSpec((1, D), lambda i: (0, 0)),
              pl.BlockSpec((1, D), lambda i: (0, 0))],
    out_specs=pl.BlockSpec((_BN, D), lambda i: (i, 0)),
    out_shape=jax.ShapeDtypeStruct((N, D), _F32),
)


def _node_init_body(x_ref, bt_ref, chg_ref, wch_ref, bch_ref, wa_ref, ba_ref,
                    out_ref):
    ch = chg_ref[:, 0:1] * wch_ref[...] + bch_ref[...]
    chw = jnp.dot(ch, wa_ref[128:128 + 16, :],
                  preferred_element_type=_F32)
    ids = lax.broadcasted_iota(jnp.int32, (1, G), 1)
    oh = (bt_ref[...] == ids).astype(_F32)
    y = (jnp.dot(x_ref[...], wa_ref[0:128, :], preferred_element_type=_F32)
         + jnp.dot(oh, chw, preferred_element_type=_F32) + ba_ref[...])
    out_ref[...] = y


_node_init = pl.pallas_call(
    _node_init_body,
    grid=(N // _BN,),
    in_specs=[pl.BlockSpec((_BN, 128), lambda i: (i, 0)),
              pl.BlockSpec((_BN, 1), lambda i: (i, 0)),
              pl.BlockSpec((G, 128), lambda i: (0, 0)),
              pl.BlockSpec((1, G), lambda i: (0, 0)),
              pl.BlockSpec((1, G), lambda i: (0, 0)),
              pl.BlockSpec((144, D), lambda i: (0, 0)),
              pl.BlockSpec((1, D), lambda i: (0, 0))],
    out_specs=pl.BlockSpec((_BN, D), lambda i: (i, 0)),
    out_shape=jax.ShapeDtypeStruct((N, D), _F32),
)


def _pool_body(x_ref, bt_ref, out_ref):
    i = pl.program_id(0)

    @pl.when(i == 0)
    def _():
        out_ref[...] = jnp.zeros_like(out_ref)

    ids = lax.broadcasted_iota(jnp.int32, (1, G), 1)
    oh = (bt_ref[...] == ids).astype(_F32)
    ones = jnp.ones((_BN, 1), _F32)
    zeros = jnp.zeros((_BN, 128 - D - 1), _F32)
    aug = jnp.concatenate([x_ref[...], ones, zeros], axis=1)
    out_ref[...] += jnp.dot(oh.T, aug, preferred_element_type=_F32)


_pool = pl.pallas_call(
    _pool_body,
    grid=(N // _BN,),
    in_specs=[pl.BlockSpec((_BN, D), lambda i: (i, 0)),
              pl.BlockSpec((_BN, 1), lambda i: (i, 0))],
    out_specs=pl.BlockSpec((G, 128), lambda i: (0, 0)),
    out_shape=jax.ShapeDtypeStruct((G, 128), _F32),
)


def _head_body(po_ref, wp1_ref, bp1_ref, wp2_ref, bp2_ref, wp3_ref, bp3_ref,
               out_ref):
    po = po_ref[...]
    gr = po[:, :D] / jnp.maximum(po[:, D:D + 1], 1.0)
    h = _sp(jnp.dot(gr, wp1_ref[...], preferred_element_type=_F32)
            + bp1_ref[...])
    h = _sp(jnp.dot(h, wp2_ref[...], preferred_element_type=_F32)
            + bp2_ref[...])
    out_ref[...] = (jnp.dot(h, wp3_ref[...], preferred_element_type=_F32)
                    + bp3_ref[...])


_head = pl.pallas_call(
    _head_body,
    grid=(1,),
    in_specs=[pl.BlockSpec((G, 128), lambda i: (0, 0)),
              pl.BlockSpec((D, 128), lambda i: (0, 0)),
              pl.BlockSpec((1, 128), lambda i: (0, 0)),
              pl.BlockSpec((128, 128), lambda i: (0, 0)),
              pl.BlockSpec((1, 128), lambda i: (0, 0)),
              pl.BlockSpec((128, 8), lambda i: (0, 0)),
              pl.BlockSpec((1, 8), lambda i: (0, 0))],
    out_specs=pl.BlockSpec((G, 8), lambda i: (0, 0)),
    out_shape=jax.ShapeDtypeStruct((G, 8), _F32),
)


def kernel(x, edge_attr, charge, params, edge_index, batch):
    p = params
    rowA, rowB = edge_index[0, :E2], edge_index[0, E2:]
    colA, colB = edge_index[1, :E2], edge_index[1, E2:]
    batch2 = batch[:, None]
    charge_b = jnp.broadcast_to(charge[:, None], (G, 128))
    zeros32 = jnp.zeros((_ZR, _DH), _F32)

    xcur = _node_init(x, batch2, charge_b, p['Wch'], p['bch'][None, :],
                      p['Wa'], p['ba'][None, :])
    pkA = edge_attr[:E2]
    pkB = edge_attr[E2:]

    for i in range(L):
        w1 = jnp.zeros((3 * D, 2 * D), _BF)
        w1 = w1.at[:, :D].set(p[f'We1_{i}'].astype(_BF))
        w1 = w1.at[0:D, D:].set(p[f'Wn1_{i}'][0:D].astype(_BF))
        b1 = jnp.concatenate([p[f'be1_{i}'], p[f'bn1_{i}']])[None, :]
        ew = (w1, b1, p[f'We2_{i}'].astype(_BF), p[f'be2_{i}'][None, :],
              p[f'Wn1_{i}'][D:2 * D].astype(_BF),
              p[f'Wn2_{i}'].astype(_BF), p[f'bn2_{i}'][None, :],
              p['Wb'], p['bb'][None, :])
        em = _edge_mlp_first if i == 0 else _edge_mlp_next
        (xrcA,) = _sc_gather(xcur, rowA, colA)
        (xrcB,) = _sc_gather(xcur, rowB, colB)
        pkA = em(xrcA, pkA, *ew)
        pkB = em(xrcB, pkB, *ew)
        (xnA,) = _sc_scatter(pkA, colA, zeros32)
        (xnB,) = _sc_scatter(pkB, colB, zeros32)
        st = _stats(xnA, xnB)
        xcur = _norm(xnA, xnB, xcur, st,
                     p[f'g_{i}'][None, :], p[f'bt_{i}'][None, :])

    po = _pool(xcur, batch2)
    out = _head(po, p['Wp1'], p['bp1'][None, :], p['Wp2'], p['bp2'][None, :],
                jnp.pad(p['Wp3'], ((0, 0), (0, 7))),
                jnp.pad(p['bp3'][None, :], ((0, 0), (0, 7))))
    return out[:, 0]
